# baseline (speedup 1.0000x reference)
import jax
import jax.numpy as jnp
from jax import lax
from jax.experimental import pallas as pl
from jax.experimental.pallas import tpu as pltpu
from jax.experimental.pallas import tpu_sc as plsc

N = 131072
D = 32
NC, NS, L = 2, 16, 16
NW = NC * NS
N_TC = 32768
N_SC = N - N_TC
SAMPLES_PER_W = N_SC // NW
CHUNK_S = 512
NCHUNK = SAMPLES_PER_W // CHUNK_S
SWEEP_S = 128
SWEEPS = CHUNK_S // SWEEP_S
NG = SWEEP_S // L
NBUF = 3
NOBUF = 2


def _sqrt16(x):
    i = lax.bitcast_convert_type(x, jnp.int32)
    y = lax.bitcast_convert_type(jnp.int32(0x5F3759DF) - (i >> 1),
                                 jnp.float32)
    for _ in range(3):
        y = y * (1.5 - 0.5 * x * y * y)
    return x * y


def _sc_body(emb_hbm, cen_hbm, out_hbm,
             cen_v, in_v0, in_v1, in_v2, in_v3, out_v0, out_v1,
             isem0, isem1, isem2, isem3, osem0, osem1):
    wid = lax.axis_index("s") * NC + lax.axis_index("c")
    sbase = wid * SAMPLES_PER_W

    in_v = (in_v0, in_v1, in_v2, in_v3)
    out_v = (out_v0, out_v1)
    isem = (isem0, isem1, isem2, isem3)
    osem = (osem0, osem1)

    pltpu.sync_copy(cen_hbm, cen_v)

    iota = lax.iota(jnp.int32, L)
    zero16 = jnp.zeros((L,), jnp.int32)
    one16 = jnp.full((L,), 1, jnp.int32)
    rots = [(iota + j) & (D - 1) for j in range(D)]

    for b in range(NBUF):
        pltpu.async_copy(emb_hbm.at[pl.ds(sbase + b * CHUNK_S, CHUNK_S)],
                         in_v[b], isem[b])

    @pl.loop(0, NCHUNK, step=NBUF)
    def _chunk_quad(c):
        for b in range(NBUF):
            cid = c + b
            ob = b & 1
            pltpu.make_async_copy(emb_hbm.at[pl.ds(0, CHUNK_S)],
                                  in_v[b], isem[b]).wait()

            @pl.when(cid >= NOBUF)
            def _():
                pltpu.make_async_copy(out_v[ob],
                                      out_hbm.at[pl.ds(0, CHUNK_S)],
                                      osem[ob]).wait()

            @pl.loop(0, SWEEPS)
            def _sweep(s):
                srow = s * SWEEP_S
                zeros = tuple(jnp.zeros((L,), jnp.float32)
                              for _ in range(2 * NG))
                rows = [iota + (srow + g * L) for g in range(NG)]

                @pl.loop(0, D, init_carry=zeros, unroll=4)
                def _feat(j, carry):
                    rot = (iota + j) & (D - 1)
                    c0 = plsc.load_gather(cen_v, [zero16, rot])
                    c1 = plsc.load_gather(cen_v, [one16, rot])
                    acc = list(carry)
                    for g in range(NG):
                        x = plsc.load_gather(in_v[b], [rows[g], rot])
                        u0 = x - c0
                        acc[2 * g] = acc[2 * g] + u0 * u0
                        u1 = x - c1
                        acc[2 * g + 1] = acc[2 * g + 1] + u1 * u1
                    return tuple(acc)

                for g in range(NG):
                    m = jnp.minimum(_feat[2 * g], _feat[2 * g + 1]) + 1e-12
                    out_v[ob][pl.ds(srow + g * L, L)] = _sqrt16(m)

            pltpu.async_copy(out_v[ob],
                             out_hbm.at[pl.ds(sbase + cid * CHUNK_S,
                                              CHUNK_S)],
                             osem[ob])

            @pl.when(cid + NBUF < NCHUNK)
            def _():
                pltpu.async_copy(
                    emb_hbm.at[pl.ds(sbase + (cid + NBUF) * CHUNK_S,
                                     CHUNK_S)],
                    in_v[b], isem[b])

    for ob in range(NOBUF):
        pltpu.make_async_copy(out_v[ob], out_hbm.at[pl.ds(0, CHUNK_S)],
                              osem[ob]).wait()


_TC_BLOCK = 8192


def _tc_body(emb_ref, cen_ref, out_ref):
    x = emb_ref[...]
    c = cen_ref[...]
    h = jnp.sum(c * c, axis=1)
    s = jax.lax.dot_general(x * x, jnp.ones((D, 1), jnp.float32),
                            (((1,), (0,)), ((), ())),
                            preferred_element_type=jnp.float32)
    bdot = jax.lax.dot_general(x, c, (((1,), (1,)), ((), ())),
                               preferred_element_type=jnp.float32)
    d2 = s - 2.0 * bdot + h[None, :]
    out_ref[...] = jnp.sqrt(jnp.min(d2, axis=1) + 1e-12)


def _tc_head(embedded, centers):
    return pl.pallas_call(
        _tc_body,
        grid=(N_TC // _TC_BLOCK,),
        in_specs=[
            pl.BlockSpec((_TC_BLOCK, D), lambda i: (i, 0)),
            pl.BlockSpec((2, D), lambda i: (0, 0)),
        ],
        out_specs=pl.BlockSpec((_TC_BLOCK,), lambda i: (i,)),
        out_shape=jax.ShapeDtypeStruct((N_TC,), jnp.float32),
    )(embedded, centers)


def kernel(embedded, centers):
    mesh = plsc.VectorSubcoreMesh(core_axis_name="c", subcore_axis_name="s",
                                  num_cores=NC, num_subcores=NS)
    run = pl.kernel(
        _sc_body,
        out_type=jax.ShapeDtypeStruct((N_SC,), jnp.float32),
        mesh=mesh,
        compiler_params=pltpu.CompilerParams(needs_layout_passes=False,
                                             use_tc_tiling_on_sc=False),
        scratch_types=[
            pltpu.VMEM((2, D), jnp.float32),
            pltpu.VMEM((CHUNK_S, D), jnp.float32),
            pltpu.VMEM((CHUNK_S, D), jnp.float32),
            pltpu.VMEM((CHUNK_S, D), jnp.float32),
            pltpu.VMEM((CHUNK_S, D), jnp.float32),
            pltpu.VMEM((CHUNK_S,), jnp.float32),
            pltpu.VMEM((CHUNK_S,), jnp.float32),
            pltpu.SemaphoreType.DMA,
            pltpu.SemaphoreType.DMA,
            pltpu.SemaphoreType.DMA,
            pltpu.SemaphoreType.DMA,
            pltpu.SemaphoreType.DMA,
            pltpu.SemaphoreType.DMA,
        ],
    )
    tc_out = _tc_head(embedded, centers)
    sc_out = run(embedded[N_TC:], centers)
    return jnp.concatenate([tc_out, sc_out])


# speedup vs baseline: 1.4904x; 1.4904x over previous
import jax
import jax.numpy as jnp
from jax import lax
from jax.experimental import pallas as pl
from jax.experimental.pallas import tpu as pltpu
from jax.experimental.pallas import tpu_sc as plsc

N = 131072
D = 32
NC, NS, L = 2, 16, 16
NW = NC * NS
SAMPLES_PER_W = N // NW
CHUNK_S = 512
NCHUNK = SAMPLES_PER_W // CHUNK_S
SWEEP_S = 128
SWEEPS = CHUNK_S // SWEEP_S
NG = SWEEP_S // L
NBUF = 4
NOBUF = 2


def _sqrt16(x):
    i = lax.bitcast_convert_type(x, jnp.int32)
    y = lax.bitcast_convert_type(jnp.int32(0x5F3759DF) - (i >> 1),
                                 jnp.float32)
    for _ in range(3):
        y = y * (1.5 - 0.5 * x * y * y)
    return x * y


def _sc_body(emb_hbm, cen_hbm, out_hbm,
             cen_v, in_v0, in_v1, in_v2, in_v3, out_v0, out_v1,
             isem0, isem1, isem2, isem3, osem0, osem1):
    wid = lax.axis_index("s") * NC + lax.axis_index("c")
    sbase = wid * SAMPLES_PER_W

    in_v = (in_v0, in_v1, in_v2, in_v3)
    out_v = (out_v0, out_v1)
    isem = (isem0, isem1, isem2, isem3)
    osem = (osem0, osem1)

    pltpu.sync_copy(cen_hbm, cen_v)

    iota = lax.iota(jnp.int32, L)
    zero16 = jnp.zeros((L,), jnp.int32)
    one16 = jnp.full((L,), 1, jnp.int32)
    rots = [(iota + j) & (D - 1) for j in range(D)]

    for b in range(NBUF):
        pltpu.async_copy(emb_hbm.at[pl.ds(sbase + b * CHUNK_S, CHUNK_S)],
                         in_v[b], isem[b])

    @pl.loop(0, NCHUNK, step=NBUF)
    def _chunk_quad(c):
        for b in range(NBUF):
            cid = c + b
            ob = b & 1
            pltpu.make_async_copy(emb_hbm.at[pl.ds(0, CHUNK_S)],
                                  in_v[b], isem[b]).wait()

            @pl.when(cid >= NOBUF)
            def _():
                pltpu.make_async_copy(out_v[ob],
                                      out_hbm.at[pl.ds(0, CHUNK_S)],
                                      osem[ob]).wait()

            @pl.loop(0, SWEEPS)
            def _sweep(s):
                srow = s * SWEEP_S
                zeros = tuple(jnp.zeros((L,), jnp.float32)
                              for _ in range(2 * NG))
                rows = [iota + (srow + g * L) for g in range(NG)]

                @pl.loop(0, D, init_carry=zeros, unroll=4)
                def _feat(j, carry):
                    rot = (iota + j) & (D - 1)
                    c0 = plsc.load_gather(cen_v, [zero16, rot])
                    c1 = plsc.load_gather(cen_v, [one16, rot])
                    acc = list(carry)
                    for g in range(NG):
                        x = plsc.load_gather(in_v[b], [rows[g], rot])
                        u0 = x - c0
                        acc[2 * g] = acc[2 * g] + u0 * u0
                        u1 = x - c1
                        acc[2 * g + 1] = acc[2 * g + 1] + u1 * u1
                    return tuple(acc)

                for g in range(NG):
                    m = jnp.minimum(_feat[2 * g], _feat[2 * g + 1]) + 1e-12
                    out_v[ob][pl.ds(srow + g * L, L)] = _sqrt16(m)

            pltpu.async_copy(out_v[ob],
                             out_hbm.at[pl.ds(sbase + cid * CHUNK_S,
                                              CHUNK_S)],
                             osem[ob])

            @pl.when(cid + NBUF < NCHUNK)
            def _():
                pltpu.async_copy(
                    emb_hbm.at[pl.ds(sbase + (cid + NBUF) * CHUNK_S,
                                     CHUNK_S)],
                    in_v[b], isem[b])

    for ob in range(NOBUF):
        pltpu.make_async_copy(out_v[ob], out_hbm.at[pl.ds(0, CHUNK_S)],
                              osem[ob]).wait()


def kernel(embedded, centers):
    mesh = plsc.VectorSubcoreMesh(core_axis_name="c", subcore_axis_name="s",
                                  num_cores=NC, num_subcores=NS)
    run = pl.kernel(
        _sc_body,
        out_type=jax.ShapeDtypeStruct((N,), jnp.float32),
        mesh=mesh,
        compiler_params=pltpu.CompilerParams(needs_layout_passes=False,
                                             use_tc_tiling_on_sc=False),
        scratch_types=[
            pltpu.VMEM((2, D), jnp.float32),
            pltpu.VMEM((CHUNK_S, D), jnp.float32),
            pltpu.VMEM((CHUNK_S, D), jnp.float32),
            pltpu.VMEM((CHUNK_S, D), jnp.float32),
            pltpu.VMEM((CHUNK_S, D), jnp.float32),
            pltpu.VMEM((CHUNK_S,), jnp.float32),
            pltpu.VMEM((CHUNK_S,), jnp.float32),
            pltpu.SemaphoreType.DMA,
            pltpu.SemaphoreType.DMA,
            pltpu.SemaphoreType.DMA,
            pltpu.SemaphoreType.DMA,
            pltpu.SemaphoreType.DMA,
            pltpu.SemaphoreType.DMA,
        ],
    )
    return run(embedded, centers)
